# baseline (speedup 1.0000x reference)
import functools

import jax
import jax.numpy as jnp
import numpy as np
from jax import lax
from jax.experimental import pallas as pl
from jax.experimental.pallas import tpu as pltpu
from jax.experimental.pallas import tpu_sc as plsc

N = 10000
E = 320000
D = 128
H = 128
C = 10
G = 64

NW = 32
K = 128
CHUNKS = 80
IBLK = 16
NIB = CHUNKS // IBLK
EPT = CHUNKS * K
EP = NW * EPT
NP = 10240
RPT = NP // 16
BLK = 1024
NBLK = NP // BLK

_Q = np.empty((H,), np.int64)
for _k in range(H // 32):
    for _t in range(16):
        _Q[32 * _k + _t] = 32 * _k + 2 * _t
        _Q[32 * _k + 16 + _t] = 32 * _k + 2 * _t + 1
_QINV = np.argsort(_Q)

def _deg_sc_body(dst_hbm, ones_hbm, zeros_hbm, out_hbm, dst_v, ones_v, acc):
    c = lax.axis_index("c")
    s = lax.axis_index("s")
    wid = s * 2 + c
    pltpu.sync_copy(dst_hbm.at[pl.ds(wid * CHUNKS, CHUNKS)], dst_v)
    pltpu.sync_copy(ones_hbm, ones_v)
    pltpu.sync_copy(zeros_hbm, acc.at[pl.ds(s * RPT, RPT)])
    plsc.subcore_barrier()

    def body(j, carry):
        pltpu.sync_copy(ones_v, acc.at[dst_v.at[j]], add=True)
        return carry

    lax.fori_loop(0, CHUNKS, body, 0)
    plsc.subcore_barrier()
    pltpu.sync_copy(acc.at[pl.ds(s * RPT, RPT)], out_hbm.at[c, pl.ds(s * RPT, RPT)])


def _scatter_sc_body(y_hbm, src_hbm, dst_hbm, zeros_hbm, out_hbm,
                     src_v, dst_v, bf0, bf1, rowsf, acc, sem0, sem1):
    c = lax.axis_index("c")
    s = lax.axis_index("s")
    wid = s * 2 + c
    pltpu.sync_copy(zeros_hbm, acc.at[pl.ds(s * RPT, RPT)])
    plsc.subcore_barrier()

    def unpack_rows(bf):
        shamt = jnp.full((16,), 16, jnp.int32)
        mask = jnp.full((16,), -65536, jnp.int32)

        def row(r, carry):
            for k in range(H // 32):
                v = bf[r, pl.ds(16 * k, 16)]
                lo = lax.bitcast_convert_type(lax.shift_left(v, shamt),
                                              jnp.float32)
                hi = lax.bitcast_convert_type(lax.bitwise_and(v, mask),
                                              jnp.float32)
                rowsf[r, pl.ds(32 * k, 16)] = lo
                rowsf[r, pl.ds(32 * k + 16, 16)] = hi
            return carry
        lax.fori_loop(0, K, row, 0)

    def pair(j, carry):
        cp0 = pltpu.async_copy(y_hbm.at[src_v.at[2 * j]], bf0, sem0)
        cp1 = pltpu.async_copy(y_hbm.at[src_v.at[2 * j + 1]], bf1, sem1)
        cp0.wait()
        unpack_rows(bf0)
        pltpu.sync_copy(rowsf, acc.at[dst_v.at[2 * j]], add=True)
        cp1.wait()
        unpack_rows(bf1)
        pltpu.sync_copy(rowsf, acc.at[dst_v.at[2 * j + 1]], add=True)
        return carry

    def iblock(b, carry):
        base = wid * CHUNKS + b * IBLK
        pltpu.sync_copy(src_hbm.at[pl.ds(base, IBLK)], src_v)
        pltpu.sync_copy(dst_hbm.at[pl.ds(base, IBLK)], dst_v)
        lax.fori_loop(0, IBLK // 2, pair, carry)
        return carry

    lax.fori_loop(0, NIB, iblock, 0)
    plsc.subcore_barrier()
    pltpu.sync_copy(acc.at[pl.ds(s * RPT, RPT)], out_hbm.at[c, pl.ds(s * RPT, RPT)])


@functools.cache
def _sc_kernels():
    mesh = plsc.VectorSubcoreMesh(core_axis_name="c", subcore_axis_name="s")
    deg = pl.kernel(
        _deg_sc_body,
        out_type=jax.ShapeDtypeStruct((2, NP, H), jnp.float32),
        mesh=mesh,
        scratch_types=[
            pltpu.VMEM((CHUNKS, K), jnp.int32),
            pltpu.VMEM((K, H), jnp.float32),
            pltpu.VMEM_SHARED((NP, H), jnp.float32),
        ],
    )
    scatter = pl.kernel(
        _scatter_sc_body,
        out_type=jax.ShapeDtypeStruct((2, NP, H), jnp.float32),
        mesh=mesh,
        compiler_params=pltpu.CompilerParams(use_tc_tiling_on_sc=False),
        scratch_types=[
            pltpu.VMEM((IBLK, K), jnp.int32),
            pltpu.VMEM((IBLK, K), jnp.int32),
            pltpu.VMEM((K, H // 2), jnp.int32),
            pltpu.VMEM((K, H // 2), jnp.int32),
            pltpu.VMEM((K, H), jnp.float32),
            pltpu.VMEM_SHARED((NP, H), jnp.float32),
            pltpu.SemaphoreType.DMA,
            pltpu.SemaphoreType.DMA,
        ],
    )
    return deg, scatter


def _tc1_body(x_ref, w_ref, wp_ref, p0_ref, p1_ref, y_ref, ybf_ref, dinv_ref):
    deg = p0_ref[:, 0:1] + p1_ref[:, 0:1] + 1.0
    dinv = lax.rsqrt(deg)
    xw = jnp.dot(x_ref[...], w_ref[...], preferred_element_type=jnp.float32)
    y_ref[...] = xw * dinv
    xwp = jnp.dot(x_ref[...], wp_ref[...], preferred_element_type=jnp.float32)
    ybf_ref[...] = (xwp * dinv).astype(jnp.bfloat16)
    dinv_ref[...] = jnp.broadcast_to(dinv, xw.shape)


def _tc1(xp, W, Wp, p0, p1):
    return pl.pallas_call(
        _tc1_body,
        grid=(NBLK,),
        in_specs=[
            pl.BlockSpec((BLK, D), lambda i: (i, 0)),
            pl.BlockSpec((D, H), lambda i: (0, 0)),
            pl.BlockSpec((D, H), lambda i: (0, 0)),
            pl.BlockSpec((BLK, H), lambda i: (i, 0)),
            pl.BlockSpec((BLK, H), lambda i: (i, 0)),
        ],
        out_specs=[
            pl.BlockSpec((BLK, H), lambda i: (i, 0)),
            pl.BlockSpec((BLK, H), lambda i: (i, 0)),
            pl.BlockSpec((BLK, H), lambda i: (i, 0)),
        ],
        out_shape=[
            jax.ShapeDtypeStruct((NP, H), jnp.float32),
            jax.ShapeDtypeStruct((NP, H), jnp.bfloat16),
            jax.ShapeDtypeStruct((NP, H), jnp.float32),
        ],
    )(xp, W, Wp, p0, p1)


def _tc2_body(z0_ref, z1_ref, yp_ref, dinv_ref, b_ref, g_ref, be_ref, w_ref,
              wp_ref, yn_ref, ynbf_ref):
    z = z0_ref[...] + z1_ref[...] + yp_ref[...]
    out = z * dinv_ref[...] + b_ref[...]
    h = jnp.maximum(out, 0.0)
    mu = jnp.mean(h, axis=-1, keepdims=True)
    d = h - mu
    var = jnp.mean(d * d, axis=-1, keepdims=True)
    hn = d * lax.rsqrt(var + 1e-5) * g_ref[...] + be_ref[...]
    yn_ref[...] = jnp.dot(hn, w_ref[...], preferred_element_type=jnp.float32) * dinv_ref[...]
    ynbf_ref[...] = (jnp.dot(hn, wp_ref[...], preferred_element_type=jnp.float32)
                     * dinv_ref[...]).astype(jnp.bfloat16)


def _tc2(z0, z1, yp, dinv, b, g, be, Wn, Wnp):
    row = pl.BlockSpec((BLK, H), lambda i: (i, 0))
    vec = pl.BlockSpec((1, H), lambda i: (0, 0))
    mat = pl.BlockSpec((H, H), lambda i: (0, 0))
    return pl.pallas_call(
        _tc2_body,
        grid=(NBLK,),
        in_specs=[row, row, row, row, vec, vec, vec, mat, mat],
        out_specs=[row, row],
        out_shape=[jax.ShapeDtypeStruct((NP, H), jnp.float32),
                   jax.ShapeDtypeStruct((NP, H), jnp.bfloat16)],
    )(z0, z1, yp, dinv, b.reshape(1, H), g.reshape(1, H), be.reshape(1, H),
      Wn, Wnp)


def _tc3_body(z0_ref, z1_ref, yp_ref, dinv_ref, b_ref, batch_ref, wc_ref, bc_ref,
              emb_ref, logp_ref, sums, cnt):
    i = pl.program_id(0)

    @pl.when(i == 0)
    def _():
        sums[...] = jnp.zeros_like(sums)
        cnt[...] = jnp.zeros_like(cnt)

    z = z0_ref[...] + z1_ref[...] + yp_ref[...]
    out3 = z * dinv_ref[...] + b_ref[...]
    ids = batch_ref[...].reshape(1, BLK)
    gid = lax.broadcasted_iota(jnp.int32, (G, BLK), 0)
    oht = (gid == ids).astype(jnp.float32)
    sums[...] += jnp.dot(oht, out3, preferred_element_type=jnp.float32)
    cnt[...] += jnp.broadcast_to(jnp.sum(oht, axis=1, keepdims=True), (G, H))

    @pl.when(i == NBLK - 1)
    def _():
        pooled = sums[...] / jnp.maximum(cnt[...], 1.0)
        emb = jnp.dot(pooled, wc_ref[...], preferred_element_type=jnp.float32) + bc_ref[...]
        mask = lax.broadcasted_iota(jnp.int32, (G, H), 1) < C
        m = jnp.max(jnp.where(mask, emb, -jnp.inf), axis=-1, keepdims=True)
        ssum = jnp.sum(jnp.where(mask, jnp.exp(emb - m), 0.0), axis=-1, keepdims=True)
        logp = emb - m - jnp.log(ssum)
        emb_ref[...] = emb[:, :C]
        logp_ref[...] = logp[:, :C]


def _tc3(z0, z1, yp, dinv, b, batch3d, Wcp, bcp):
    row = pl.BlockSpec((BLK, H), lambda i: (i, 0))
    return pl.pallas_call(
        _tc3_body,
        grid=(NBLK,),
        in_specs=[row, row, row, row,
                  pl.BlockSpec((1, H), lambda i: (0, 0)),
                  pl.BlockSpec((1, 1, BLK), lambda i: (i, 0, 0)),
                  pl.BlockSpec((H, H), lambda i: (0, 0)),
                  pl.BlockSpec((1, H), lambda i: (0, 0))],
        out_specs=[pl.BlockSpec((G, C), lambda i: (0, 0)),
                   pl.BlockSpec((G, C), lambda i: (0, 0))],
        out_shape=[jax.ShapeDtypeStruct((G, C), jnp.float32),
                   jax.ShapeDtypeStruct((G, C), jnp.float32)],
        scratch_shapes=[pltpu.VMEM((G, H), jnp.float32),
                        pltpu.VMEM((G, H), jnp.float32)],
    )(z0, z1, yp, dinv, b, batch3d, Wcp, bcp)


def kernel(x, edge_index, batch, W1, b1, g1, be1, W2, b2, g2, be2, W3, b3, Wc, bc):
    pad = EP - E
    srcp = jnp.concatenate([edge_index[0], jnp.zeros((pad,), jnp.int32)])
    dstp = jnp.concatenate([edge_index[1], jnp.full((pad,), N, jnp.int32)])
    src2d = srcp.reshape(NW * CHUNKS, K)
    dst2d = dstp.reshape(NW * CHUNKS, K)
    xp = jnp.zeros((NP, D), jnp.float32).at[:N].set(x)
    batch3d = jnp.concatenate(
        [batch, jnp.full((NP - N,), G, jnp.int32)]).reshape(NBLK, 1, BLK)
    ones_h = jnp.ones((K, H), jnp.float32)
    zeros_h = jnp.zeros((RPT, H), jnp.float32)
    Wcp = jnp.zeros((H, H), jnp.float32).at[:, :C].set(Wc)
    bcp = jnp.zeros((1, H), jnp.float32).at[0, :C].set(bc)

    W1p = W1[:, _QINV]
    W2p = W2[:, _QINV]
    W3p = W3[:, _QINV]
    def _pack(ybf):
        return lax.bitcast_convert_type(
            ybf.reshape(NP, H // 2, 2), jnp.int32)

    deg_sc, scatter_sc = _sc_kernels()
    degp = deg_sc(dst2d, ones_h, zeros_h)
    y1, y1bf, dinv = _tc1(xp, W1, W1p, degp[0], degp[1])
    zz = scatter_sc(_pack(y1bf), src2d, dst2d, zeros_h)
    y2, y2bf = _tc2(zz[0], zz[1], y1, dinv, b1, g1, be1, W2, W2p)
    zz = scatter_sc(_pack(y2bf), src2d, dst2d, zeros_h)
    y3, y3bf = _tc2(zz[0], zz[1], y2, dinv, b2, g2, be2, W3, W3p)
    zz = scatter_sc(_pack(y3bf), src2d, dst2d, zeros_h)
    b3r = b3.reshape(1, H)
    emb, logp = _tc3(zz[0], zz[1], y3, dinv, b3r, batch3d, Wcp, bcp)
    return (emb, logp)


# speedup vs baseline: 1.5008x; 1.5008x over previous
import functools

import jax
import jax.numpy as jnp
import numpy as np
from jax import lax
from jax.experimental import pallas as pl
from jax.experimental.pallas import tpu as pltpu
from jax.experimental.pallas import tpu_sc as plsc

N = 10000
E = 320000
D = 128
H = 128
C = 10
G = 64

NW = 32
K = 128
CHUNKS = 80
IBLK = 16
NIB = CHUNKS // IBLK
EPT = CHUNKS * K
EP = NW * EPT
NP = 10240
RPT = NP // 16
BLK = 1024
NBLK = NP // BLK

_Q = np.empty((H,), np.int64)
for _k in range(H // 32):
    for _t in range(16):
        _Q[32 * _k + _t] = 32 * _k + 2 * _t
        _Q[32 * _k + 16 + _t] = 32 * _k + 2 * _t + 1
_QINV = np.argsort(_Q)

def _deg_sc_body(dst_hbm, ones_hbm, zeros_hbm, out_hbm, dst_v, ones_v, acc):
    c = lax.axis_index("c")
    s = lax.axis_index("s")
    wid = s * 2 + c
    pltpu.sync_copy(dst_hbm.at[pl.ds(wid * CHUNKS, CHUNKS)], dst_v)
    pltpu.sync_copy(ones_hbm, ones_v)
    pltpu.sync_copy(zeros_hbm, acc.at[pl.ds(s * RPT, RPT)])
    plsc.subcore_barrier()

    def body(j, carry):
        pltpu.sync_copy(ones_v, acc.at[dst_v.at[j]], add=True)
        return carry

    lax.fori_loop(0, CHUNKS, body, 0)
    plsc.subcore_barrier()
    pltpu.sync_copy(acc.at[pl.ds(s * RPT, RPT)], out_hbm.at[c, pl.ds(s * RPT, RPT)])


def _scatter_sc_body(y_hbm, src_hbm, dst_hbm, zeros_hbm, out_hbm,
                     src_v, dst_v, bf0, bf1, rowsf, acc, sem0, sem1):
    c = lax.axis_index("c")
    s = lax.axis_index("s")
    wid = s * 2 + c
    pltpu.sync_copy(src_hbm.at[pl.ds(wid * CHUNKS, CHUNKS)], src_v)
    pltpu.sync_copy(zeros_hbm, acc.at[pl.ds(s * RPT, RPT)])
    plsc.subcore_barrier()

    def unpack_rows(bf):
        shamt = jnp.full((16,), 16, jnp.int32)
        mask = jnp.full((16,), -65536, jnp.int32)

        def row(r, carry):
            for k in range(H // 32):
                v = bf[r, pl.ds(16 * k, 16)]
                lo = lax.bitcast_convert_type(lax.shift_left(v, shamt),
                                              jnp.float32)
                hi = lax.bitcast_convert_type(lax.bitwise_and(v, mask),
                                              jnp.float32)
                rowsf[r, pl.ds(32 * k, 16)] = lo
                rowsf[r, pl.ds(32 * k + 16, 16)] = hi
            return carry
        lax.fori_loop(0, K, row, 0)

    def gather(e, buf, sem):
        return pltpu.async_copy(y_hbm.at[src_v.at[e]], buf, sem)

    gather(0, bf0, sem0)

    def iblock(b, carry):
        base = wid * CHUNKS + b * IBLK
        pltpu.sync_copy(dst_hbm.at[pl.ds(base, IBLK)], dst_v)

        def pair(j, carry2):
            e = b * IBLK + 2 * j
            gather(e + 1, bf1, sem1)
            pltpu.make_async_copy(y_hbm.at[src_v.at[e]], bf0, sem0).wait()
            unpack_rows(bf0)
            pltpu.sync_copy(rowsf, acc.at[dst_v.at[2 * j]], add=True)

            @pl.when(e + 2 < CHUNKS)
            def _():
                gather(e + 2, bf0, sem0)

            pltpu.make_async_copy(y_hbm.at[src_v.at[e + 1]], bf1, sem1).wait()
            unpack_rows(bf1)
            pltpu.sync_copy(rowsf, acc.at[dst_v.at[2 * j + 1]], add=True)
            return carry2

        lax.fori_loop(0, IBLK // 2, pair, carry)
        return carry

    lax.fori_loop(0, NIB, iblock, 0)
    plsc.subcore_barrier()
    pltpu.sync_copy(acc.at[pl.ds(s * RPT, RPT)], out_hbm.at[c, pl.ds(s * RPT, RPT)])


@functools.cache
def _sc_kernels():
    mesh = plsc.VectorSubcoreMesh(core_axis_name="c", subcore_axis_name="s")
    deg = pl.kernel(
        _deg_sc_body,
        out_type=jax.ShapeDtypeStruct((2, NP, H), jnp.float32),
        mesh=mesh,
        scratch_types=[
            pltpu.VMEM((CHUNKS, K), jnp.int32),
            pltpu.VMEM((K, H), jnp.float32),
            pltpu.VMEM_SHARED((NP, H), jnp.float32),
        ],
    )
    scatter = pl.kernel(
        _scatter_sc_body,
        out_type=jax.ShapeDtypeStruct((2, NP, H), jnp.float32),
        mesh=mesh,
        compiler_params=pltpu.CompilerParams(use_tc_tiling_on_sc=False),
        scratch_types=[
            pltpu.VMEM((CHUNKS, K), jnp.int32),
            pltpu.VMEM((IBLK, K), jnp.int32),
            pltpu.VMEM((K, H // 2), jnp.int32),
            pltpu.VMEM((K, H // 2), jnp.int32),
            pltpu.VMEM((K, H), jnp.float32),
            pltpu.VMEM_SHARED((NP, H), jnp.float32),
            pltpu.SemaphoreType.DMA,
            pltpu.SemaphoreType.DMA,
        ],
    )
    return deg, scatter


def _tc1_body(x_ref, w_ref, wp_ref, p0_ref, p1_ref, y_ref, ybf_ref, dinv_ref):
    deg = p0_ref[:, 0:1] + p1_ref[:, 0:1] + 1.0
    dinv = lax.rsqrt(deg)
    xw = jnp.dot(x_ref[...], w_ref[...], preferred_element_type=jnp.float32)
    y_ref[...] = xw * dinv
    xwp = jnp.dot(x_ref[...], wp_ref[...], preferred_element_type=jnp.float32)
    ybf_ref[...] = (xwp * dinv).astype(jnp.bfloat16)
    dinv_ref[...] = jnp.broadcast_to(dinv, xw.shape)


def _tc1(xp, W, Wp, p0, p1):
    return pl.pallas_call(
        _tc1_body,
        grid=(NBLK,),
        in_specs=[
            pl.BlockSpec((BLK, D), lambda i: (i, 0)),
            pl.BlockSpec((D, H), lambda i: (0, 0)),
            pl.BlockSpec((D, H), lambda i: (0, 0)),
            pl.BlockSpec((BLK, H), lambda i: (i, 0)),
            pl.BlockSpec((BLK, H), lambda i: (i, 0)),
        ],
        out_specs=[
            pl.BlockSpec((BLK, H), lambda i: (i, 0)),
            pl.BlockSpec((BLK, H), lambda i: (i, 0)),
            pl.BlockSpec((BLK, H), lambda i: (i, 0)),
        ],
        out_shape=[
            jax.ShapeDtypeStruct((NP, H), jnp.float32),
            jax.ShapeDtypeStruct((NP, H), jnp.bfloat16),
            jax.ShapeDtypeStruct((NP, H), jnp.float32),
        ],
    )(xp, W, Wp, p0, p1)


def _tc2_body(z0_ref, z1_ref, yp_ref, dinv_ref, b_ref, g_ref, be_ref, w_ref,
              wp_ref, yn_ref, ynbf_ref):
    z = z0_ref[...] + z1_ref[...] + yp_ref[...]
    out = z * dinv_ref[...] + b_ref[...]
    h = jnp.maximum(out, 0.0)
    mu = jnp.mean(h, axis=-1, keepdims=True)
    d = h - mu
    var = jnp.mean(d * d, axis=-1, keepdims=True)
    hn = d * lax.rsqrt(var + 1e-5) * g_ref[...] + be_ref[...]
    yn_ref[...] = jnp.dot(hn, w_ref[...], preferred_element_type=jnp.float32) * dinv_ref[...]
    ynbf_ref[...] = (jnp.dot(hn, wp_ref[...], preferred_element_type=jnp.float32)
                     * dinv_ref[...]).astype(jnp.bfloat16)


def _tc2(z0, z1, yp, dinv, b, g, be, Wn, Wnp):
    row = pl.BlockSpec((BLK, H), lambda i: (i, 0))
    vec = pl.BlockSpec((1, H), lambda i: (0, 0))
    mat = pl.BlockSpec((H, H), lambda i: (0, 0))
    return pl.pallas_call(
        _tc2_body,
        grid=(NBLK,),
        in_specs=[row, row, row, row, vec, vec, vec, mat, mat],
        out_specs=[row, row],
        out_shape=[jax.ShapeDtypeStruct((NP, H), jnp.float32),
                   jax.ShapeDtypeStruct((NP, H), jnp.bfloat16)],
    )(z0, z1, yp, dinv, b.reshape(1, H), g.reshape(1, H), be.reshape(1, H),
      Wn, Wnp)


def _tc3_body(z0_ref, z1_ref, yp_ref, dinv_ref, b_ref, batch_ref, wc_ref, bc_ref,
              emb_ref, logp_ref, sums, cnt):
    i = pl.program_id(0)

    @pl.when(i == 0)
    def _():
        sums[...] = jnp.zeros_like(sums)
        cnt[...] = jnp.zeros_like(cnt)

    z = z0_ref[...] + z1_ref[...] + yp_ref[...]
    out3 = z * dinv_ref[...] + b_ref[...]
    ids = batch_ref[...].reshape(1, BLK)
    gid = lax.broadcasted_iota(jnp.int32, (G, BLK), 0)
    oht = (gid == ids).astype(jnp.float32)
    sums[...] += jnp.dot(oht, out3, preferred_element_type=jnp.float32)
    cnt[...] += jnp.broadcast_to(jnp.sum(oht, axis=1, keepdims=True), (G, H))

    @pl.when(i == NBLK - 1)
    def _():
        pooled = sums[...] / jnp.maximum(cnt[...], 1.0)
        emb = jnp.dot(pooled, wc_ref[...], preferred_element_type=jnp.float32) + bc_ref[...]
        mask = lax.broadcasted_iota(jnp.int32, (G, H), 1) < C
        m = jnp.max(jnp.where(mask, emb, -jnp.inf), axis=-1, keepdims=True)
        ssum = jnp.sum(jnp.where(mask, jnp.exp(emb - m), 0.0), axis=-1, keepdims=True)
        logp = emb - m - jnp.log(ssum)
        emb_ref[...] = emb[:, :C]
        logp_ref[...] = logp[:, :C]


def _tc3(z0, z1, yp, dinv, b, batch3d, Wcp, bcp):
    row = pl.BlockSpec((BLK, H), lambda i: (i, 0))
    return pl.pallas_call(
        _tc3_body,
        grid=(NBLK,),
        in_specs=[row, row, row, row,
                  pl.BlockSpec((1, H), lambda i: (0, 0)),
                  pl.BlockSpec((1, 1, BLK), lambda i: (i, 0, 0)),
                  pl.BlockSpec((H, H), lambda i: (0, 0)),
                  pl.BlockSpec((1, H), lambda i: (0, 0))],
        out_specs=[pl.BlockSpec((G, C), lambda i: (0, 0)),
                   pl.BlockSpec((G, C), lambda i: (0, 0))],
        out_shape=[jax.ShapeDtypeStruct((G, C), jnp.float32),
                   jax.ShapeDtypeStruct((G, C), jnp.float32)],
        scratch_shapes=[pltpu.VMEM((G, H), jnp.float32),
                        pltpu.VMEM((G, H), jnp.float32)],
    )(z0, z1, yp, dinv, b, batch3d, Wcp, bcp)


def kernel(x, edge_index, batch, W1, b1, g1, be1, W2, b2, g2, be2, W3, b3, Wc, bc):
    pad = EP - E
    srcp = jnp.concatenate([edge_index[0], jnp.zeros((pad,), jnp.int32)])
    dstp = jnp.concatenate([edge_index[1], jnp.full((pad,), N, jnp.int32)])
    src2d = srcp.reshape(NW * CHUNKS, K)
    dst2d = dstp.reshape(NW * CHUNKS, K)
    xp = jnp.zeros((NP, D), jnp.float32).at[:N].set(x)
    batch3d = jnp.concatenate(
        [batch, jnp.full((NP - N,), G, jnp.int32)]).reshape(NBLK, 1, BLK)
    ones_h = jnp.ones((K, H), jnp.float32)
    zeros_h = jnp.zeros((RPT, H), jnp.float32)
    Wcp = jnp.zeros((H, H), jnp.float32).at[:, :C].set(Wc)
    bcp = jnp.zeros((1, H), jnp.float32).at[0, :C].set(bc)

    W1p = W1[:, _QINV]
    W2p = W2[:, _QINV]
    W3p = W3[:, _QINV]
    def _pack(ybf):
        return lax.bitcast_convert_type(
            ybf.reshape(NP, H // 2, 2), jnp.int32)

    deg_sc, scatter_sc = _sc_kernels()
    degp = deg_sc(dst2d, ones_h, zeros_h)
    y1, y1bf, dinv = _tc1(xp, W1, W1p, degp[0], degp[1])
    zz = scatter_sc(_pack(y1bf), src2d, dst2d, zeros_h)
    y2, y2bf = _tc2(zz[0], zz[1], y1, dinv, b1, g1, be1, W2, W2p)
    zz = scatter_sc(_pack(y2bf), src2d, dst2d, zeros_h)
    y3, y3bf = _tc2(zz[0], zz[1], y2, dinv, b2, g2, be2, W3, W3p)
    zz = scatter_sc(_pack(y3bf), src2d, dst2d, zeros_h)
    b3r = b3.reshape(1, H)
    emb, logp = _tc3(zz[0], zz[1], y3, dinv, b3r, batch3d, Wcp, bcp)
    return (emb, logp)
